# baseline (speedup 1.0000x reference)
import functools
import jax
import jax.numpy as jnp
from jax.experimental import pallas as pl
from jax.experimental.pallas import tpu as pltpu

N, L, K = 320000, 128, 16
BLK = 6400
G = BLK // 8
NBLK = N // BLK
DENOM = float(L * K * (K - 1))


def _body(t_ref, phi_ref, out_ref, acc_ref, cnt_ref):
    i = pl.program_id(0)

    @pl.when(i == 0)
    def _init():
        acc_ref[...] = jnp.zeros_like(acc_ref)
        cnt_ref[...] = jnp.zeros_like(cnt_ref)

    m = t_ref[...].astype(jnp.float32)
    cnt_ref[...] += jnp.sum(m, axis=0, keepdims=True)

    p = jax.lax.dot_general(
        m, phi_ref[...], (((0,), (0,)), ((), ())),
        preferred_element_type=jnp.float32)
    acc = acc_ref[...]
    for r in range(8):
        acc = acc + p[16 * r:16 * (r + 1), 128 * r:128 * (r + 1)]
    acc_ref[...] = acc

    @pl.when(i == NBLK - 1)
    def _epilogue():
        s = acc_ref[...]
        cl = cnt_ref[...]
        c_row = (cl[:, 0:K] + cl[:, K:2 * K] + cl[:, 2 * K:3 * K]
                 + cl[:, 3 * K:4 * K] + cl[:, 4 * K:5 * K]
                 + cl[:, 5 * K:6 * K] + cl[:, 6 * K:7 * K]
                 + cl[:, 7 * K:8 * K])
        rows = jax.lax.broadcasted_iota(jnp.int32, (K, K), 0)
        cols = jax.lax.broadcasted_iota(jnp.int32, (K, K), 1)
        eye = (rows == cols).astype(jnp.float32)
        c_col = jax.lax.dot_general(
            eye, c_row, (((1,), (1,)), ((), ())),
            preferred_element_type=jnp.float32)
        gram_s = jax.lax.dot_general(
            s, s, (((1,), (1,)), ((), ())),
            preferred_element_type=jnp.float32)
        gram = gram_s / (c_col * c_row)
        sq_col = jnp.sum(gram * eye, axis=1, keepdims=True)
        sq_row = jnp.sum(gram * eye, axis=0, keepdims=True)
        d2 = sq_col + sq_row - 2.0 * gram
        dist = jnp.sqrt(jnp.maximum(d2, 0.0))
        offdiag = (rows != cols).astype(jnp.float32)
        out_ref[0, 0] = jnp.sum(dist * offdiag) / DENOM


@jax.jit
def kernel(phi_x, t):
    t2 = jnp.reshape(t, (N // 8, 128))
    phi2 = jnp.reshape(phi_x, (N // 8, 1024))
    out = pl.pallas_call(
        _body,
        grid=(NBLK,),
        in_specs=[
            pl.BlockSpec((G, 128), lambda i: (i, 0)),
            pl.BlockSpec((G, 1024), lambda i: (i, 0)),
        ],
        out_specs=pl.BlockSpec(memory_space=pltpu.SMEM),
        out_shape=jax.ShapeDtypeStruct((1, 1), jnp.float32),
        scratch_shapes=[
            pltpu.VMEM((K, L), jnp.float32),
            pltpu.VMEM((1, 128), jnp.float32),
        ],
    )(t2, phi2)
    return out[0, 0]


# speedup vs baseline: 1.6901x; 1.6901x over previous
import functools
import jax
import jax.numpy as jnp
from jax.experimental import pallas as pl
from jax.experimental.pallas import tpu as pltpu

N, L, K = 320000, 128, 16
BLK = 6400
G = BLK // 8
NBLK = N // BLK
DENOM = float(L * K * (K - 1))


def _start_phi_dmas(phi_hbm, cat_ref, sem, blk, slot):
    for r in range(8):
        pltpu.make_async_copy(
            phi_hbm.at[pl.ds(blk * G, G), r, :],
            cat_ref.at[slot, :, pl.ds(128 * r, 128)],
            sem.at[slot, r],
        ).start()


def _wait_phi_dmas(phi_hbm, cat_ref, sem, blk, slot):
    for r in range(8):
        pltpu.make_async_copy(
            phi_hbm.at[pl.ds(blk * G, G), r, :],
            cat_ref.at[slot, :, pl.ds(128 * r, 128)],
            sem.at[slot, r],
        ).wait()


def _body(t_ref, phi_hbm, out_ref, acc_ref, cnt_ref, cat_ref, sem):
    i = pl.program_id(0)
    slot = jax.lax.rem(i, 2)

    @pl.when(i == 0)
    def _init():
        acc_ref[...] = jnp.zeros_like(acc_ref)
        cnt_ref[...] = jnp.zeros_like(cnt_ref)
        _start_phi_dmas(phi_hbm, cat_ref, sem, 0, 0)

    @pl.when(i + 1 < NBLK)
    def _prefetch():
        _start_phi_dmas(phi_hbm, cat_ref, sem, i + 1, 1 - slot)

    _wait_phi_dmas(phi_hbm, cat_ref, sem, i, slot)

    m = t_ref[...].astype(jnp.float32)
    cnt_ref[...] += jnp.sum(m, axis=0, keepdims=True)

    p = jax.lax.dot_general(
        m, cat_ref[slot], (((0,), (0,)), ((), ())),
        preferred_element_type=jnp.float32)
    acc = acc_ref[...]
    for r in range(8):
        acc = acc + p[16 * r:16 * (r + 1), 128 * r:128 * (r + 1)]
    acc_ref[...] = acc

    @pl.when(i == NBLK - 1)
    def _epilogue():
        s = acc_ref[...]
        cl = cnt_ref[...]
        c_row = (cl[:, 0:K] + cl[:, K:2 * K] + cl[:, 2 * K:3 * K]
                 + cl[:, 3 * K:4 * K] + cl[:, 4 * K:5 * K]
                 + cl[:, 5 * K:6 * K] + cl[:, 6 * K:7 * K]
                 + cl[:, 7 * K:8 * K])
        rows = jax.lax.broadcasted_iota(jnp.int32, (K, K), 0)
        cols = jax.lax.broadcasted_iota(jnp.int32, (K, K), 1)
        eye = (rows == cols).astype(jnp.float32)
        c_col = jax.lax.dot_general(
            eye, c_row, (((1,), (1,)), ((), ())),
            preferred_element_type=jnp.float32)
        gram_s = jax.lax.dot_general(
            s, s, (((1,), (1,)), ((), ())),
            preferred_element_type=jnp.float32)
        gram = gram_s / (c_col * c_row)
        sq_col = jnp.sum(gram * eye, axis=1, keepdims=True)
        sq_row = jnp.sum(gram * eye, axis=0, keepdims=True)
        d2 = sq_col + sq_row - 2.0 * gram
        dist = jnp.sqrt(jnp.maximum(d2, 0.0))
        offdiag = (rows != cols).astype(jnp.float32)
        out_ref[0, 0] = jnp.sum(dist * offdiag) / DENOM


@jax.jit
def kernel(phi_x, t):
    t2 = jnp.reshape(t, (N // 8, 128))
    phi3 = jnp.reshape(phi_x, (N // 8, 8, L))
    out = pl.pallas_call(
        _body,
        grid=(NBLK,),
        in_specs=[
            pl.BlockSpec((G, 128), lambda i: (i, 0)),
            pl.BlockSpec(memory_space=pl.ANY),
        ],
        out_specs=pl.BlockSpec(memory_space=pltpu.SMEM),
        out_shape=jax.ShapeDtypeStruct((1, 1), jnp.float32),
        scratch_shapes=[
            pltpu.VMEM((K, L), jnp.float32),
            pltpu.VMEM((1, 128), jnp.float32),
            pltpu.VMEM((2, G, 1024), jnp.float32),
            pltpu.SemaphoreType.DMA((2, 8)),
        ],
    )(t2, phi3)
    return out[0, 0]


# speedup vs baseline: 5.4341x; 3.2153x over previous
import functools
import jax
import jax.numpy as jnp
from jax.experimental import pallas as pl
from jax.experimental.pallas import tpu as pltpu

N, L, K = 320000, 128, 16
BLK = 6400
NBLK = N // BLK
DENOM = float(L * K * (K - 1))


def _body(tt_ref, phi_ref, out_ref, acc_ref, cnt_ref):
    i = pl.program_id(0)

    @pl.when(i == 0)
    def _init():
        acc_ref[...] = jnp.zeros_like(acc_ref)
        cnt_ref[...] = jnp.zeros_like(cnt_ref)

    mt = tt_ref[...].astype(jnp.float32)
    cnt_ref[...] += jnp.sum(mt, axis=1, keepdims=True)
    acc_ref[...] += jax.lax.dot_general(
        mt, phi_ref[...], (((1,), (0,)), ((), ())),
        preferred_element_type=jnp.float32)

    @pl.when(i == NBLK - 1)
    def _epilogue():
        s = acc_ref[...]
        c_col = cnt_ref[...]
        rows = jax.lax.broadcasted_iota(jnp.int32, (K, K), 0)
        cols = jax.lax.broadcasted_iota(jnp.int32, (K, K), 1)
        eye = (rows == cols).astype(jnp.float32)
        c_row = jax.lax.dot_general(
            c_col, eye, (((0,), (0,)), ((), ())),
            preferred_element_type=jnp.float32)
        gram_s = jax.lax.dot_general(
            s, s, (((1,), (1,)), ((), ())),
            preferred_element_type=jnp.float32)
        gram = gram_s / (c_col * c_row)
        sq_col = jnp.sum(gram * eye, axis=1, keepdims=True)
        sq_row = jnp.sum(gram * eye, axis=0, keepdims=True)
        d2 = sq_col + sq_row - 2.0 * gram
        dist = jnp.sqrt(jnp.maximum(d2, 0.0))
        offdiag = (rows != cols).astype(jnp.float32)
        out_ref[0, 0] = jnp.sum(dist * offdiag) / DENOM


@jax.jit
def kernel(phi_x, t):
    tt = jnp.transpose(t)
    out = pl.pallas_call(
        _body,
        grid=(NBLK,),
        in_specs=[
            pl.BlockSpec((K, BLK), lambda i: (0, i)),
            pl.BlockSpec((BLK, L), lambda i: (i, 0)),
        ],
        out_specs=pl.BlockSpec(memory_space=pltpu.SMEM),
        out_shape=jax.ShapeDtypeStruct((1, 1), jnp.float32),
        scratch_shapes=[
            pltpu.VMEM((K, L), jnp.float32),
            pltpu.VMEM((K, 1), jnp.float32),
        ],
    )(tt, phi_x)
    return out[0, 0]


# speedup vs baseline: 6.3858x; 1.1751x over previous
import functools
import jax
import jax.numpy as jnp
from jax.experimental import pallas as pl
from jax.experimental.pallas import tpu as pltpu

N, L, K = 320000, 128, 16
BLK = 16000
NBLK = N // BLK
DENOM = float(L * K * (K - 1))


def _body(tt_ref, phi_ref, out_ref, acc_ref, cnt_ref):
    i = pl.program_id(0)

    @pl.when(i == 0)
    def _init():
        acc_ref[...] = jnp.zeros_like(acc_ref)
        cnt_ref[...] = jnp.zeros_like(cnt_ref)

    mt = tt_ref[...].astype(jnp.float32)
    cnt_ref[...] += jnp.sum(mt, axis=1, keepdims=True)
    acc_ref[...] += jax.lax.dot_general(
        mt, phi_ref[...], (((1,), (0,)), ((), ())),
        preferred_element_type=jnp.float32)

    @pl.when(i == NBLK - 1)
    def _epilogue():
        s = acc_ref[...]
        c_col = cnt_ref[...]
        rows = jax.lax.broadcasted_iota(jnp.int32, (K, K), 0)
        cols = jax.lax.broadcasted_iota(jnp.int32, (K, K), 1)
        eye = (rows == cols).astype(jnp.float32)
        c_row = jax.lax.dot_general(
            c_col, eye, (((0,), (0,)), ((), ())),
            preferred_element_type=jnp.float32)
        gram_s = jax.lax.dot_general(
            s, s, (((1,), (1,)), ((), ())),
            preferred_element_type=jnp.float32)
        gram = gram_s / (c_col * c_row)
        sq_col = jnp.sum(gram * eye, axis=1, keepdims=True)
        sq_row = jnp.sum(gram * eye, axis=0, keepdims=True)
        d2 = sq_col + sq_row - 2.0 * gram
        dist = jnp.sqrt(jnp.maximum(d2, 0.0))
        offdiag = (rows != cols).astype(jnp.float32)
        out_ref[0, 0] = jnp.sum(dist * offdiag) / DENOM


@jax.jit
def kernel(phi_x, t):
    tt = jnp.transpose(t)
    out = pl.pallas_call(
        _body,
        grid=(NBLK,),
        in_specs=[
            pl.BlockSpec((K, BLK), lambda i: (0, i)),
            pl.BlockSpec((BLK, L), lambda i: (i, 0)),
        ],
        out_specs=pl.BlockSpec(memory_space=pltpu.SMEM),
        out_shape=jax.ShapeDtypeStruct((1, 1), jnp.float32),
        scratch_shapes=[
            pltpu.VMEM((K, L), jnp.float32),
            pltpu.VMEM((K, 1), jnp.float32),
        ],
    )(tt, phi_x)
    return out[0, 0]
